# baseline (speedup 1.0000x reference)
import functools

import jax
import jax.numpy as jnp
from jax.experimental import pallas as pl
from jax.experimental.pallas import tpu as pltpu

_TM = 512
_NEG = -1e30


def _router_w0(x, wr, br):
    logits = jnp.dot(x, wr, preferred_element_type=jnp.float32) + br
    m1 = jnp.max(logits, axis=-1, keepdims=True)
    col = jax.lax.broadcasted_iota(jnp.int32, logits.shape, 1)
    fpos = jnp.min(jnp.where(logits == m1, col, logits.shape[1]),
                   axis=-1, keepdims=True)
    m2 = jnp.max(jnp.where(col == fpos, _NEG, logits), axis=-1, keepdims=True)
    return jax.nn.sigmoid(m1 - m2)


def _expert_kernel(x_ref, wr_ref, br_ref, w1_hbm, b1_ref, w2_hbm, b2_ref,
                   *rest, expert, first_expert):
    if first_expert:
        out_ref, w1_vmem, w2_vmem, sem1, sem2 = rest
    else:
        prev_ref, out_ref, w1_vmem, w2_vmem, sem1, sem2 = rest
    first = pl.program_id(0) == 0
    half1 = w1_vmem.shape[0] // 2
    half2 = w2_vmem.shape[0] // 2

    cp1a = pltpu.make_async_copy(w1_hbm.at[expert, :half1],
                                 w1_vmem.at[:half1], sem1.at[0])
    cp1b = pltpu.make_async_copy(w1_hbm.at[expert, half1:],
                                 w1_vmem.at[half1:], sem1.at[1])
    cp2a = pltpu.make_async_copy(w2_hbm.at[expert, :half2],
                                 w2_vmem.at[:half2], sem2.at[0])
    cp2b = pltpu.make_async_copy(w2_hbm.at[expert, half2:],
                                 w2_vmem.at[half2:], sem2.at[1])

    @pl.when(first)
    def _start():
        cp1a.start()
        cp1b.start()
        cp2a.start()
        cp2b.start()
        cp1a.wait()
        cp1b.wait()

    x = x_ref[...]
    w0 = _router_w0(x, wr_ref[...], br_ref[...])
    h = jnp.maximum(jnp.dot(x, w1_vmem[...],
                            preferred_element_type=jnp.float32)
                    + b1_ref[0], 0.0)

    @pl.when(first)
    def _wait_w2():
        cp2a.wait()
        cp2b.wait()

    o = jnp.dot(h, w2_vmem[...], preferred_element_type=jnp.float32)
    if first_expert:
        out_ref[...] = w0 * (o + b2_ref[0])
    else:
        out_ref[...] = rest[0][...] + (1.0 - w0) * (o + b2_ref[0])


def kernel(inputs, Wr, br, W1, b1, W2, b2):
    B, S, D = inputs.shape
    T = B * S
    E = Wr.shape[1]
    Hid = W1.shape[2]
    x = inputs.reshape(T, D)
    brr = br.reshape(1, E)
    b1r = b1.reshape(b1.shape[0], 1, Hid)
    b2r = b2.reshape(b2.shape[0], 1, D)

    def specs(e, with_prev):
        s = [
            pl.BlockSpec((_TM, D), lambda i: (i, 0)),
            pl.BlockSpec((D, E), lambda i: (0, 0)),
            pl.BlockSpec((1, E), lambda i: (0, 0)),
            pl.BlockSpec(memory_space=pl.ANY),
            pl.BlockSpec((1, 1, Hid), lambda i: (e, 0, 0)),
            pl.BlockSpec(memory_space=pl.ANY),
            pl.BlockSpec((1, 1, D), lambda i: (e, 0, 0)),
        ]
        if with_prev:
            s.append(pl.BlockSpec((_TM, D), lambda i: (i, 0)))
        return s

    scratch = [
        pltpu.VMEM((D, Hid), jnp.float32),
        pltpu.VMEM((Hid, D), jnp.float32),
        pltpu.SemaphoreType.DMA((2,)),
        pltpu.SemaphoreType.DMA((2,)),
    ]
    grid = (T // _TM,)
    out_sd = jax.ShapeDtypeStruct((T, D), jnp.float32)
    part = pl.pallas_call(
        functools.partial(_expert_kernel, expert=0, first_expert=True),
        grid=grid,
        in_specs=specs(0, False),
        out_specs=pl.BlockSpec((_TM, D), lambda i: (i, 0)),
        out_shape=out_sd,
        scratch_shapes=scratch,
    )(x, Wr, brr, W1, b1r, W2, b2r)
    out = pl.pallas_call(
        functools.partial(_expert_kernel, expert=1, first_expert=False),
        grid=grid,
        in_specs=specs(1, True),
        out_specs=pl.BlockSpec((_TM, D), lambda i: (i, 0)),
        out_shape=out_sd,
        scratch_shapes=scratch,
        input_output_aliases={7: 0},
    )(x, Wr, brr, W1, b1r, W2, b2r, part)
    return out.reshape(B, S, D)


# speedup vs baseline: 1.0279x; 1.0279x over previous
import functools

import jax
import jax.numpy as jnp
from jax.experimental import pallas as pl
from jax.experimental.pallas import tpu as pltpu

_TM = 512
_NEG = -1e30


def _router_w0(x, wr, br):
    logits = jnp.dot(x, wr, preferred_element_type=jnp.float32) + br
    m1 = jnp.max(logits, axis=-1, keepdims=True)
    col = jax.lax.broadcasted_iota(jnp.int32, logits.shape, 1)
    fpos = jnp.min(jnp.where(logits == m1, col, logits.shape[1]),
                   axis=-1, keepdims=True)
    m2 = jnp.max(jnp.where(col == fpos, _NEG, logits), axis=-1, keepdims=True)
    return jax.nn.sigmoid(m1 - m2)


def _expert_kernel(x_ref, wr_ref, br_ref, w1_hbm, b1_ref, w2_hbm, b2_ref,
                   *rest, expert, first_expert):
    if first_expert:
        out_ref, w1_vmem, w2_vmem, sem1, sem2 = rest
    else:
        prev_ref, out_ref, w1_vmem, w2_vmem, sem1, sem2 = rest
    first = pl.program_id(0) == 0

    cp1 = pltpu.make_async_copy(w1_hbm.at[expert], w1_vmem, sem1)
    cp2 = pltpu.make_async_copy(w2_hbm.at[expert], w2_vmem, sem2)

    @pl.when(first)
    def _start():
        cp1.start()
        cp2.start()
        cp1.wait()

    x = x_ref[...]
    w0 = _router_w0(x, wr_ref[...], br_ref[...])
    h = jnp.maximum(jnp.dot(x, w1_vmem[...],
                            preferred_element_type=jnp.float32)
                    + b1_ref[0], 0.0)

    @pl.when(first)
    def _wait_w2():
        cp2.wait()

    o = jnp.dot(h, w2_vmem[...], preferred_element_type=jnp.float32)
    if first_expert:
        out_ref[...] = w0 * (o + b2_ref[0])
    else:
        out_ref[...] = rest[0][...] + (1.0 - w0) * (o + b2_ref[0])


def kernel(inputs, Wr, br, W1, b1, W2, b2):
    B, S, D = inputs.shape
    T = B * S
    E = Wr.shape[1]
    Hid = W1.shape[2]
    x = inputs.reshape(T, D)
    brr = br.reshape(1, E)
    b1r = b1.reshape(b1.shape[0], 1, Hid)
    b2r = b2.reshape(b2.shape[0], 1, D)

    def specs(e, with_prev):
        s = [
            pl.BlockSpec((_TM, D), lambda i: (i, 0)),
            pl.BlockSpec((D, E), lambda i: (0, 0)),
            pl.BlockSpec((1, E), lambda i: (0, 0)),
            pl.BlockSpec(memory_space=pl.ANY),
            pl.BlockSpec((1, 1, Hid), lambda i: (e, 0, 0)),
            pl.BlockSpec(memory_space=pl.ANY),
            pl.BlockSpec((1, 1, D), lambda i: (e, 0, 0)),
        ]
        if with_prev:
            s.append(pl.BlockSpec((_TM, D), lambda i: (i, 0)))
        return s

    scratch = [
        pltpu.VMEM((D, Hid), jnp.float32),
        pltpu.VMEM((Hid, D), jnp.float32),
        pltpu.SemaphoreType.DMA,
        pltpu.SemaphoreType.DMA,
    ]
    grid = (T // _TM,)
    out_sd = jax.ShapeDtypeStruct((T, D), jnp.float32)
    part = pl.pallas_call(
        functools.partial(_expert_kernel, expert=0, first_expert=True),
        grid=grid,
        in_specs=specs(0, False),
        out_specs=pl.BlockSpec((_TM, D), lambda i: (i, 0)),
        out_shape=out_sd,
        scratch_shapes=scratch,
    )(x, Wr, brr, W1, b1r, W2, b2r)
    out = pl.pallas_call(
        functools.partial(_expert_kernel, expert=1, first_expert=False),
        grid=grid,
        in_specs=specs(1, True),
        out_specs=pl.BlockSpec((_TM, D), lambda i: (i, 0)),
        out_shape=out_sd,
        scratch_shapes=scratch,
        input_output_aliases={7: 0},
    )(x, Wr, brr, W1, b1r, W2, b2r, part)
    return out.reshape(B, S, D)
